# baseline (speedup 1.0000x reference)
import math

import jax
import jax.numpy as jnp
from jax import lax
from jax.experimental import pallas as pl
from jax.experimental.pallas import tpu as pltpu
import jax.experimental.pallas.tpu_sc as plsc

_N = 10000
_E = 320000
_F = 128
_R = 20
_CUT = 5.0

_BN = 1000
_BE = 3200
_CH = 40
_NW = 32
_EPW = _E // _NW
_EPT = _E // 16
_NPAD = 10240
_RPT = _NPAD // 16

def _pack(a, b):
    ab = a.astype(jnp.bfloat16).astype(jnp.float32)
    bb = b.astype(jnp.bfloat16).astype(jnp.float32)
    ai = jax.lax.bitcast_convert_type(ab, jnp.uint32) >> 16
    bi = (jax.lax.bitcast_convert_type(bb, jnp.uint32) >> 16) << 16
    return jax.lax.bitcast_convert_type(ai | bi, jnp.int32)


def _tables_body(s_ref, w1_ref, b1_ref, w2_ref, b2_ref, vt_ref, tbl_ref):
    h = jnp.dot(s_ref[...], w1_ref[...], preferred_element_type=jnp.float32)
    h = h + b1_ref[...]
    h = h * jax.nn.sigmoid(h)
    phi = jnp.dot(h, w2_ref[...], preferred_element_type=jnp.float32) + b2_ref[...]
    phi1 = phi[:, : _F]
    phi2 = phi[:, _F : 2 * _F]
    phi3 = phi[:, 2 * _F :]
    p0 = phi1 * vt_ref[0]
    p1 = phi1 * vt_ref[1]
    p2 = phi1 * vt_ref[2]
    tbl_ref[:, 0:64] = _pack(p0[:, 0:64], p2[:, 64:128])
    tbl_ref[:, 64:128] = _pack(p0[:, 64:128], phi3[:, 0:64])
    tbl_ref[:, 128:192] = _pack(p1[:, 0:64], phi3[:, 64:128])
    tbl_ref[:, 192:256] = _pack(p1[:, 64:128], phi2[:, 0:64])
    tbl_ref[:, 256:320] = _pack(p2[:, 0:64], phi2[:, 64:128])
    tbl_ref[:, 320:384] = jnp.zeros((s_ref.shape[0], 64), jnp.int32)


def _tables(s, w1, b1, w2, b2, vt, interpret=False):
    grid = (_N // _BN,)
    return pl.pallas_call(
        _tables_body,
        grid=grid,
        in_specs=[
            pl.BlockSpec((_BN, _F), lambda i: (i, 0)),
            pl.BlockSpec((_F, _F), lambda i: (0, 0)),
            pl.BlockSpec((1, _F), lambda i: (0, 0)),
            pl.BlockSpec((_F, 3 * _F), lambda i: (0, 0)),
            pl.BlockSpec((1, 3 * _F), lambda i: (0, 0)),
            pl.BlockSpec((3, _BN, _F), lambda i: (0, i, 0)),
        ],
        out_specs=pl.BlockSpec((_BN, 384), lambda i: (i, 0)),
        out_shape=jax.ShapeDtypeStruct((_N, 384), jnp.int32),
        interpret=interpret,
    )(s, w1, b1, w2, b2, vt)


def _rbf_body(n128_ref, nvec_ref, ph_ref, m_ref, hm_ref, mb_ref, out_ref):
    n128 = n128_ref[...]
    q = jnp.sin(n128 * nvec_ref[...] + ph_ref[...])
    out_ref[...] = q * (m_ref[...] / n128 + hm_ref[...]) + mb_ref[...]


def _rbf(n128, nvec, ph, m, hm, mb, interpret=False):
    grid = (n128.shape[0] * 4 // _BE,)
    const = lambda i: (0, 0)
    return pl.pallas_call(
        _rbf_body,
        grid=grid,
        in_specs=[
            pl.BlockSpec((_BE // 4, 128), lambda i: (i, 0)),
            pl.BlockSpec((1, 128), const),
            pl.BlockSpec((1, 128), const),
            pl.BlockSpec((1, 128), const),
            pl.BlockSpec((1, 128), const),
            pl.BlockSpec((1, 128), const),
        ],
        out_specs=pl.BlockSpec((_BE // 4, 128), lambda i: (i, 0)),
        out_shape=jax.ShapeDtypeStruct(n128.shape, jnp.float32),
        interpret=interpret,
    )(n128, nvec, ph, m, hm, mb)


def _edge_body(g_ref, rs_ref, norm_ref, diff_ref, waug_ref, upd_ref):
    en = norm_ref[...]
    rs = rs_ref[...]
    fc = rs[:, 20:21]
    w0 = jnp.dot(rs, waug_ref[...], preferred_element_type=jnp.float32)
    w = w0 * fc
    gi = jax.lax.bitcast_convert_type(g_ref[:, 0:320], jnp.uint32)
    lo = jax.lax.bitcast_convert_type(gi << 16, jnp.float32)
    hi = jax.lax.bitcast_convert_type((gi >> 16) << 16, jnp.float32)
    p0 = lo[:, 0:128]
    p1 = lo[:, 128:256]
    p2 = jnp.concatenate([lo[:, 256:320], hi[:, 0:64]], axis=1)
    phi3 = hi[:, 64:192]
    phi2 = hi[:, 192:320]
    w1 = w[:, : _F]
    w2 = w[:, _F : 2 * _F]
    w3 = w[:, 2 * _F :]
    t3 = w3 * phi3
    u = diff_ref[...] / en
    upd_ref[:, 0 * _F : 1 * _F] = w1 * p0 + t3 * u[:, 0:1]
    upd_ref[:, 1 * _F : 2 * _F] = w1 * p1 + t3 * u[:, 1:2]
    upd_ref[:, 2 * _F : 3 * _F] = w1 * p2 + t3 * u[:, 2:3]
    upd_ref[:, 3 * _F : 4 * _F] = w2 * phi2


def _edges(g, rs, norm2d, diff, waug, interpret=False):
    grid = (g.shape[0] // _BE,)
    const = lambda i: (0, 0)
    return pl.pallas_call(
        _edge_body,
        grid=grid,
        in_specs=[
            pl.BlockSpec((_BE, 384), lambda i: (i, 0)),
            pl.BlockSpec((_BE, 32), lambda i: (i, 0)),
            pl.BlockSpec((_BE, 1), lambda i: (i, 0)),
            pl.BlockSpec((_BE, 3), lambda i: (i, 0)),
            pl.BlockSpec((32, 3 * _F), const),
        ],
        out_specs=pl.BlockSpec((_BE, 4 * _F), lambda i: (i, 0)),
        out_shape=jax.ShapeDtypeStruct((g.shape[0], 4 * _F), jnp.float32),
        interpret=interpret,
    )(g, rs, norm2d, diff, waug)


_GCH = 80
_GNB = 4
_EH = _E
_EPWH = _EH // _NW


def _gather_body(tbl_hbm, src_hbm, g_hbm, *rest):
    idxs = rest[0:_GNB]
    rows = rest[_GNB : 2 * _GNB]
    isems = rest[2 * _GNB : 3 * _GNB]
    gsems = rest[3 * _GNB : 4 * _GNB]
    wsems = rest[4 * _GNB : 5 * _GNB]
    c = lax.axis_index("c")
    sub = lax.axis_index("s")
    wid = sub * 2 + c
    base0 = wid * _EPWH
    nch = _EPWH // _GCH

    def start_idx(j, b):
        base = base0 + j * _GCH
        pltpu.async_copy(src_hbm.at[pl.ds(base, _GCH)], idxs[b], isems[b])

    def start_gather(j, b):
        base = base0 + j * _GCH
        pltpu.make_async_copy(
            src_hbm.at[pl.ds(base, _GCH)], idxs[b], isems[b]
        ).wait()
        pltpu.async_copy(tbl_hbm.at[idxs[b]], rows[b], gsems[b])

    def wait_w(b):
        pltpu.make_async_copy(
            rows[b], g_hbm.at[pl.ds(base0, _GCH)], wsems[b]
        ).wait()

    for b in range(_GNB):
        start_idx(b, b)
    for b in range(_GNB):
        start_gather(b, b)

    def body(u, carry):
        for b in range(_GNB):
            j = u * _GNB + b
            base = base0 + j * _GCH
            pltpu.make_async_copy(tbl_hbm.at[idxs[b]], rows[b], gsems[b]).wait()
            pltpu.async_copy(rows[b], g_hbm.at[pl.ds(base, _GCH)], wsems[b])

            @pl.when(j + _GNB < nch)
            def _():
                wait_w(b)
                start_idx(j + _GNB, b)

        for b in range(_GNB):
            j = u * _GNB + b

            @pl.when(j + _GNB < nch)
            def _():
                start_gather(j + _GNB, b)

        return carry

    lax.fori_loop(0, nch // _GNB, body, 0)
    for j in range(_GNB * (nch // _GNB), nch):
        b = j % _GNB
        base = base0 + j * _GCH
        pltpu.make_async_copy(tbl_hbm.at[idxs[b]], rows[b], gsems[b]).wait()
        pltpu.async_copy(rows[b], g_hbm.at[pl.ds(base, _GCH)], wsems[b])
    for b in range(_GNB):
        wait_w(b)


def _gather(tbl, src):
    mesh = plsc.VectorSubcoreMesh(core_axis_name="c", subcore_axis_name="s")
    fn = pl.kernel(
        _gather_body,
        out_type=jax.ShapeDtypeStruct((_EH, 384), jnp.int32),
        mesh=mesh,
        scratch_types=(
            [pltpu.VMEM((_GCH,), jnp.int32) for _ in range(_GNB)]
            + [pltpu.VMEM((_GCH, 384), jnp.int32) for _ in range(_GNB)]
            + [pltpu.SemaphoreType.DMA for _ in range(3 * _GNB)]
        ),
    )
    return fn(tbl, src)


_SNB = 5
_ZR = 16


def _scatter_body(upd_hbm, dst_hbm, out_hbm, *rest):
    idxs = rest[0:_SNB]
    datas = rest[_SNB : 2 * _SNB]
    zero_v = rest[2 * _SNB]
    isems = rest[2 * _SNB + 1 : 3 * _SNB + 1]
    dsems = rest[3 * _SNB + 1 : 4 * _SNB + 1]
    ssems = rest[4 * _SNB + 1 : 5 * _SNB + 1]
    acc = rest[5 * _SNB + 1]
    c = lax.axis_index("c")
    sub = lax.axis_index("s")
    nch = _EPT // _CH

    def zbody(i, carry):
        zero_v[i // (_F // 16), pl.ds((i % (_F // 16)) * 16, 16)] = jnp.zeros(
            (16,), jnp.float32
        )
        return carry

    lax.fori_loop(0, (_ZR * _F) // 16, zbody, 0)

    for p in range(2):
        for z in range(_RPT // _ZR):
            pltpu.sync_copy(zero_v, acc.at[pl.ds(sub * _RPT + z * _ZR, _ZR)])
        plsc.subcore_barrier()

        col = (2 * p + c) * _F

        for hh, upd_hbm in ((0, upd_hbm),):

            def start(j, b):
                base = sub * (_EH // 16) + j * _CH
                pltpu.async_copy(
                    dst_hbm.at[pl.ds(hh * _EH + base, _CH)], idxs[b], isems[b]
                )
                pltpu.async_copy(
                    upd_hbm.at[pl.ds(base, _CH), pl.ds(col, _F)], datas[b],
                    dsems[b],
                )

            def wait_s(b):
                pltpu.make_async_copy(datas[b], acc.at[idxs[b]], ssems[b]).wait()

            nchh = (_EH // 16) // _CH

            for b in range(_SNB):
                start(b, b)

            def body(u, carry):
                for b in range(_SNB):
                    j = u * _SNB + b
                    base = sub * (_EH // 16) + j * _CH
                    pltpu.make_async_copy(
                        dst_hbm.at[pl.ds(hh * _EH + base, _CH)], idxs[b],
                        isems[b],
                    ).wait()
                    pltpu.make_async_copy(
                        upd_hbm.at[pl.ds(base, _CH), pl.ds(col, _F)], datas[b],
                        dsems[b],
                    ).wait()
                    pltpu.async_copy(
                        datas[b], acc.at[idxs[b]], ssems[b], add=True
                    )
                    jn = j + _SNB

                    @pl.when(jn < nchh)
                    def _():
                        wait_s(b)
                        start(jn, b)

                return carry

            lax.fori_loop(0, nchh // _SNB, body, 0)
            for b in range(_SNB):
                wait_s(b)
        plsc.subcore_barrier()

        pltpu.sync_copy(
            acc.at[pl.ds(sub * _RPT, _RPT)],
            out_hbm.at[p, pl.ds(sub * _RPT, _RPT), pl.ds(c * _F, _F)],
        )
        plsc.subcore_barrier()


def _scatter(upd, dst):
    mesh = plsc.VectorSubcoreMesh(core_axis_name="c", subcore_axis_name="s")
    fn = pl.kernel(
        _scatter_body,
        out_type=jax.ShapeDtypeStruct((2, _NPAD, 2 * _F), jnp.float32),
        mesh=mesh,
        scratch_types=(
            [pltpu.VMEM((_CH,), jnp.int32) for _ in range(_SNB)]
            + [pltpu.VMEM((_CH, _F), jnp.float32) for _ in range(_SNB)]
            + [pltpu.VMEM((_ZR, _F), jnp.float32)]
            + [pltpu.SemaphoreType.DMA for _ in range(3 * _SNB)]
            + [pltpu.VMEM_SHARED((_NPAD, _F), jnp.float32)]
        ),
    )
    return fn(upd, dst)


def kernel(v, s, edges_indices, edges_diff, edges_norm, W1, b1, W2, b2, Wr, br):
    src = edges_indices[:, 1].astype(jnp.int32)
    dst = edges_indices[:, 0].astype(jnp.int32)
    vt = jnp.transpose(v, (2, 0, 1))

    r32 = jnp.arange(32, dtype=jnp.float32)
    nv32 = jnp.where(r32 < _R, (r32 + 1.0) * (math.pi / _CUT),
                     jnp.where(r32 == _R, math.pi / _CUT, 0.0))
    ph32 = jnp.where(r32 == _R, 0.5 * math.pi, 0.0)
    m32 = (r32 < _R).astype(jnp.float32)
    hm32 = 0.5 * (r32 == _R).astype(jnp.float32)
    mb32 = jnp.where(r32 == _R, 0.5, 0.0) + (r32 == _R + 1).astype(jnp.float32)
    tile4 = lambda a: jnp.tile(a, 4).reshape(1, 128)
    nvec, ph, m, hm, mb = (tile4(a) for a in (nv32, ph32, m32, hm32, mb32))
    waug = jnp.concatenate(
        [Wr, jnp.zeros((1, 3 * _F), jnp.float32), br.reshape(1, 3 * _F),
         jnp.zeros((32 - _R - 2, 3 * _F), jnp.float32)], axis=0)
    n128 = jnp.broadcast_to(
        edges_norm.reshape(_E, 1), (_E, 32)).reshape(_E // 4, 128)

    tbl = _tables(s, W1, b1.reshape(1, _F), W2, b2.reshape(1, 3 * _F), vt)
    norm2d = edges_norm.reshape(_E, 1)
    g = _gather(tbl, src)
    rs = _rbf(n128, nvec, ph, m, hm, mb).reshape(_E, 32)
    upd = _edges(g, rs, norm2d, edges_diff, waug)
    out = _scatter(upd, dst)[:, : _N]

    dv = jnp.stack([out[0, :, : _F], out[0, :, _F :], out[1, :, : _F]], axis=-1)
    ds = out[1, :, _F :]
    return (dv, ds)


# speedup vs baseline: 1.0048x; 1.0048x over previous
import math

import jax
import jax.numpy as jnp
from jax import lax
from jax.experimental import pallas as pl
from jax.experimental.pallas import tpu as pltpu
import jax.experimental.pallas.tpu_sc as plsc

_N = 10000
_E = 320000
_F = 128
_R = 20
_CUT = 5.0

_BN = 1000
_BE = 4000
_CH = 40
_NW = 32
_EPW = _E // _NW
_EPT = _E // 16
_NPAD = 10240
_RPT = _NPAD // 16

def _pack(a, b):
    ab = a.astype(jnp.bfloat16).astype(jnp.float32)
    bb = b.astype(jnp.bfloat16).astype(jnp.float32)
    ai = jax.lax.bitcast_convert_type(ab, jnp.uint32) >> 16
    bi = (jax.lax.bitcast_convert_type(bb, jnp.uint32) >> 16) << 16
    return jax.lax.bitcast_convert_type(ai | bi, jnp.int32)


def _tables_body(s_ref, w1_ref, b1_ref, w2_ref, b2_ref, vt_ref, tbl_ref):
    h = jnp.dot(s_ref[...], w1_ref[...], preferred_element_type=jnp.float32)
    h = h + b1_ref[...]
    h = h * jax.nn.sigmoid(h)
    phi = jnp.dot(h, w2_ref[...], preferred_element_type=jnp.float32) + b2_ref[...]
    phi1 = phi[:, : _F]
    phi2 = phi[:, _F : 2 * _F]
    phi3 = phi[:, 2 * _F :]
    p0 = phi1 * vt_ref[0]
    p1 = phi1 * vt_ref[1]
    p2 = phi1 * vt_ref[2]
    tbl_ref[:, 0:64] = _pack(p0[:, 0:64], p2[:, 64:128])
    tbl_ref[:, 64:128] = _pack(p0[:, 64:128], phi3[:, 0:64])
    tbl_ref[:, 128:192] = _pack(p1[:, 0:64], phi3[:, 64:128])
    tbl_ref[:, 192:256] = _pack(p1[:, 64:128], phi2[:, 0:64])
    tbl_ref[:, 256:320] = _pack(p2[:, 0:64], phi2[:, 64:128])
    tbl_ref[:, 320:384] = jnp.zeros((s_ref.shape[0], 64), jnp.int32)


def _tables(s, w1, b1, w2, b2, vt, interpret=False):
    grid = (_N // _BN,)
    return pl.pallas_call(
        _tables_body,
        grid=grid,
        in_specs=[
            pl.BlockSpec((_BN, _F), lambda i: (i, 0)),
            pl.BlockSpec((_F, _F), lambda i: (0, 0)),
            pl.BlockSpec((1, _F), lambda i: (0, 0)),
            pl.BlockSpec((_F, 3 * _F), lambda i: (0, 0)),
            pl.BlockSpec((1, 3 * _F), lambda i: (0, 0)),
            pl.BlockSpec((3, _BN, _F), lambda i: (0, i, 0)),
        ],
        out_specs=pl.BlockSpec((_BN, 384), lambda i: (i, 0)),
        out_shape=jax.ShapeDtypeStruct((_N, 384), jnp.int32),
        interpret=interpret,
    )(s, w1, b1, w2, b2, vt)


def _rbf_body(n128_ref, nvec_ref, ph_ref, m_ref, hm_ref, mb_ref, out_ref):
    n128 = n128_ref[...]
    q = jnp.sin(n128 * nvec_ref[...] + ph_ref[...])
    out_ref[...] = q * (m_ref[...] / n128 + hm_ref[...]) + mb_ref[...]


def _rbf(n128, nvec, ph, m, hm, mb, interpret=False):
    grid = (n128.shape[0] * 4 // _BE,)
    const = lambda i: (0, 0)
    return pl.pallas_call(
        _rbf_body,
        grid=grid,
        in_specs=[
            pl.BlockSpec((_BE // 4, 128), lambda i: (i, 0)),
            pl.BlockSpec((1, 128), const),
            pl.BlockSpec((1, 128), const),
            pl.BlockSpec((1, 128), const),
            pl.BlockSpec((1, 128), const),
            pl.BlockSpec((1, 128), const),
        ],
        out_specs=pl.BlockSpec((_BE // 4, 128), lambda i: (i, 0)),
        out_shape=jax.ShapeDtypeStruct(n128.shape, jnp.float32),
        interpret=interpret,
    )(n128, nvec, ph, m, hm, mb)


def _edge_body(g_ref, rs_ref, norm_ref, diff_ref, waug_ref, upd_ref):
    en = norm_ref[...]
    rs = rs_ref[...]
    fc = rs[:, 20:21]
    w0 = jnp.dot(rs, waug_ref[...], preferred_element_type=jnp.float32)
    w = w0 * fc
    gi = jax.lax.bitcast_convert_type(g_ref[:, 0:320], jnp.uint32)
    lo = jax.lax.bitcast_convert_type(gi << 16, jnp.float32)
    hi = jax.lax.bitcast_convert_type((gi >> 16) << 16, jnp.float32)
    p0 = lo[:, 0:128]
    p1 = lo[:, 128:256]
    p2 = jnp.concatenate([lo[:, 256:320], hi[:, 0:64]], axis=1)
    phi3 = hi[:, 64:192]
    phi2 = hi[:, 192:320]
    w1 = w[:, : _F]
    w2 = w[:, _F : 2 * _F]
    w3 = w[:, 2 * _F :]
    t3 = w3 * phi3
    u = diff_ref[...] / en
    upd_ref[:, 0 * _F : 1 * _F] = w1 * p0 + t3 * u[:, 0:1]
    upd_ref[:, 1 * _F : 2 * _F] = w1 * p1 + t3 * u[:, 1:2]
    upd_ref[:, 2 * _F : 3 * _F] = w1 * p2 + t3 * u[:, 2:3]
    upd_ref[:, 3 * _F : 4 * _F] = w2 * phi2


def _edges(g, rs, norm2d, diff, waug, interpret=False):
    grid = (g.shape[0] // _BE,)
    const = lambda i: (0, 0)
    return pl.pallas_call(
        _edge_body,
        grid=grid,
        in_specs=[
            pl.BlockSpec((_BE, 384), lambda i: (i, 0)),
            pl.BlockSpec((_BE, 32), lambda i: (i, 0)),
            pl.BlockSpec((_BE, 1), lambda i: (i, 0)),
            pl.BlockSpec((_BE, 3), lambda i: (i, 0)),
            pl.BlockSpec((32, 3 * _F), const),
        ],
        out_specs=pl.BlockSpec((_BE, 4 * _F), lambda i: (i, 0)),
        out_shape=jax.ShapeDtypeStruct((g.shape[0], 4 * _F), jnp.float32),
        interpret=interpret,
    )(g, rs, norm2d, diff, waug)


_GCH = 80
_GNB = 4
_EH = _E
_EPWH = _EH // _NW


def _gather_body(tbl_hbm, src_hbm, g_hbm, *rest):
    idxs = rest[0:_GNB]
    rows = rest[_GNB : 2 * _GNB]
    isems = rest[2 * _GNB : 3 * _GNB]
    gsems = rest[3 * _GNB : 4 * _GNB]
    wsems = rest[4 * _GNB : 5 * _GNB]
    c = lax.axis_index("c")
    sub = lax.axis_index("s")
    wid = sub * 2 + c
    base0 = wid * _EPWH
    nch = _EPWH // _GCH

    def start_idx(j, b):
        base = base0 + j * _GCH
        pltpu.async_copy(src_hbm.at[pl.ds(base, _GCH)], idxs[b], isems[b])

    def start_gather(j, b):
        base = base0 + j * _GCH
        pltpu.make_async_copy(
            src_hbm.at[pl.ds(base, _GCH)], idxs[b], isems[b]
        ).wait()
        pltpu.async_copy(tbl_hbm.at[idxs[b]], rows[b], gsems[b])

    def wait_w(b):
        pltpu.make_async_copy(
            rows[b], g_hbm.at[pl.ds(base0, _GCH)], wsems[b]
        ).wait()

    for b in range(_GNB):
        start_idx(b, b)
    for b in range(_GNB):
        start_gather(b, b)

    def body(u, carry):
        for b in range(_GNB):
            j = u * _GNB + b
            base = base0 + j * _GCH
            pltpu.make_async_copy(tbl_hbm.at[idxs[b]], rows[b], gsems[b]).wait()
            pltpu.async_copy(rows[b], g_hbm.at[pl.ds(base, _GCH)], wsems[b])

            @pl.when(j + _GNB < nch)
            def _():
                wait_w(b)
                start_idx(j + _GNB, b)

        for b in range(_GNB):
            j = u * _GNB + b

            @pl.when(j + _GNB < nch)
            def _():
                start_gather(j + _GNB, b)

        return carry

    lax.fori_loop(0, nch // _GNB, body, 0)
    for j in range(_GNB * (nch // _GNB), nch):
        b = j % _GNB
        base = base0 + j * _GCH
        pltpu.make_async_copy(tbl_hbm.at[idxs[b]], rows[b], gsems[b]).wait()
        pltpu.async_copy(rows[b], g_hbm.at[pl.ds(base, _GCH)], wsems[b])
    for b in range(_GNB):
        wait_w(b)


def _gather(tbl, src):
    mesh = plsc.VectorSubcoreMesh(core_axis_name="c", subcore_axis_name="s")
    fn = pl.kernel(
        _gather_body,
        out_type=jax.ShapeDtypeStruct((_EH, 384), jnp.int32),
        mesh=mesh,
        scratch_types=(
            [pltpu.VMEM((_GCH,), jnp.int32) for _ in range(_GNB)]
            + [pltpu.VMEM((_GCH, 384), jnp.int32) for _ in range(_GNB)]
            + [pltpu.SemaphoreType.DMA for _ in range(3 * _GNB)]
        ),
    )
    return fn(tbl, src)


_SNB = 5
_ZR = 16


def _scatter_body(upd_hbm, dst_hbm, out_hbm, *rest):
    idxs = rest[0:_SNB]
    datas = rest[_SNB : 2 * _SNB]
    zero_v = rest[2 * _SNB]
    isems = rest[2 * _SNB + 1 : 3 * _SNB + 1]
    dsems = rest[3 * _SNB + 1 : 4 * _SNB + 1]
    ssems = rest[4 * _SNB + 1 : 5 * _SNB + 1]
    acc = rest[5 * _SNB + 1]
    c = lax.axis_index("c")
    sub = lax.axis_index("s")
    nch = _EPT // _CH

    def zbody(i, carry):
        zero_v[i // (_F // 16), pl.ds((i % (_F // 16)) * 16, 16)] = jnp.zeros(
            (16,), jnp.float32
        )
        return carry

    lax.fori_loop(0, (_ZR * _F) // 16, zbody, 0)

    for p in range(2):
        for z in range(_RPT // _ZR):
            pltpu.sync_copy(zero_v, acc.at[pl.ds(sub * _RPT + z * _ZR, _ZR)])
        plsc.subcore_barrier()

        col = (2 * p + c) * _F

        for hh, upd_hbm in ((0, upd_hbm),):

            def start(j, b):
                base = sub * (_EH // 16) + j * _CH
                pltpu.async_copy(
                    dst_hbm.at[pl.ds(hh * _EH + base, _CH)], idxs[b], isems[b]
                )
                pltpu.async_copy(
                    upd_hbm.at[pl.ds(base, _CH), pl.ds(col, _F)], datas[b],
                    dsems[b],
                )

            def wait_s(b):
                pltpu.make_async_copy(datas[b], acc.at[idxs[b]], ssems[b]).wait()

            nchh = (_EH // 16) // _CH

            for b in range(_SNB):
                start(b, b)

            def body(u, carry):
                for b in range(_SNB):
                    j = u * _SNB + b
                    base = sub * (_EH // 16) + j * _CH
                    pltpu.make_async_copy(
                        dst_hbm.at[pl.ds(hh * _EH + base, _CH)], idxs[b],
                        isems[b],
                    ).wait()
                    pltpu.make_async_copy(
                        upd_hbm.at[pl.ds(base, _CH), pl.ds(col, _F)], datas[b],
                        dsems[b],
                    ).wait()
                    pltpu.async_copy(
                        datas[b], acc.at[idxs[b]], ssems[b], add=True
                    )
                    jn = j + _SNB

                    @pl.when(jn < nchh)
                    def _():
                        wait_s(b)
                        start(jn, b)

                return carry

            lax.fori_loop(0, nchh // _SNB, body, 0)
            for b in range(_SNB):
                wait_s(b)
        plsc.subcore_barrier()

        pltpu.sync_copy(
            acc.at[pl.ds(sub * _RPT, _RPT)],
            out_hbm.at[p, pl.ds(sub * _RPT, _RPT), pl.ds(c * _F, _F)],
        )
        plsc.subcore_barrier()


def _scatter(upd, dst):
    mesh = plsc.VectorSubcoreMesh(core_axis_name="c", subcore_axis_name="s")
    fn = pl.kernel(
        _scatter_body,
        out_type=jax.ShapeDtypeStruct((2, _NPAD, 2 * _F), jnp.float32),
        mesh=mesh,
        scratch_types=(
            [pltpu.VMEM((_CH,), jnp.int32) for _ in range(_SNB)]
            + [pltpu.VMEM((_CH, _F), jnp.float32) for _ in range(_SNB)]
            + [pltpu.VMEM((_ZR, _F), jnp.float32)]
            + [pltpu.SemaphoreType.DMA for _ in range(3 * _SNB)]
            + [pltpu.VMEM_SHARED((_NPAD, _F), jnp.float32)]
        ),
    )
    return fn(upd, dst)


def kernel(v, s, edges_indices, edges_diff, edges_norm, W1, b1, W2, b2, Wr, br):
    src = edges_indices[:, 1].astype(jnp.int32)
    dst = edges_indices[:, 0].astype(jnp.int32)
    vt = jnp.transpose(v, (2, 0, 1))

    r32 = jnp.arange(32, dtype=jnp.float32)
    nv32 = jnp.where(r32 < _R, (r32 + 1.0) * (math.pi / _CUT),
                     jnp.where(r32 == _R, math.pi / _CUT, 0.0))
    ph32 = jnp.where(r32 == _R, 0.5 * math.pi, 0.0)
    m32 = (r32 < _R).astype(jnp.float32)
    hm32 = 0.5 * (r32 == _R).astype(jnp.float32)
    mb32 = jnp.where(r32 == _R, 0.5, 0.0) + (r32 == _R + 1).astype(jnp.float32)
    tile4 = lambda a: jnp.tile(a, 4).reshape(1, 128)
    nvec, ph, m, hm, mb = (tile4(a) for a in (nv32, ph32, m32, hm32, mb32))
    waug = jnp.concatenate(
        [Wr, jnp.zeros((1, 3 * _F), jnp.float32), br.reshape(1, 3 * _F),
         jnp.zeros((32 - _R - 2, 3 * _F), jnp.float32)], axis=0)
    n128 = jnp.broadcast_to(
        edges_norm.reshape(_E, 1), (_E, 32)).reshape(_E // 4, 128)

    tbl = _tables(s, W1, b1.reshape(1, _F), W2, b2.reshape(1, 3 * _F), vt)
    norm2d = edges_norm.reshape(_E, 1)
    g = _gather(tbl, src)
    rs = _rbf(n128, nvec, ph, m, hm, mb).reshape(_E, 32)
    upd = _edges(g, rs, norm2d, edges_diff, waug)
    out = _scatter(upd, dst)[:, : _N]

    dv = jnp.stack([out[0, :, : _F], out[0, :, _F :], out[1, :, : _F]], axis=-1)
    ds = out[1, :, _F :]
    return (dv, ds)
